# Initial kernel scaffold; baseline (speedup 1.0000x reference)
import jax
import jax.numpy as jnp
from jax.experimental import pallas as pl


def kernel(x, mask_indices):
    raise NotImplementedError("write your pallas kernel here")


# speedup vs baseline: 3.7596x; 3.7596x over previous
import functools

import jax
import jax.numpy as jnp
from jax import lax
from jax.experimental import pallas as pl
from jax.experimental.pallas import tpu as pltpu
from jax.experimental.pallas import tpu_sc as plsc

_LANES = 16


def _mask_body(idx_hbm, out_hbm, idx_v, mask_v):
    n_pad = idx_v.shape[0]
    n_cols = mask_v.shape[0]

    @pl.when(jnp.logical_and(lax.axis_index("c") == 0, lax.axis_index("s") == 0))
    def _():
        pltpu.sync_copy(idx_hbm, idx_v)
        ones = jnp.ones((_LANES,), jnp.float32)

        def init(i, carry):
            mask_v[pl.ds(i * _LANES, _LANES)] = ones
            return carry

        lax.fori_loop(0, n_cols // _LANES, init, 0)

        zeros = jnp.zeros((_LANES,), jnp.float32)

        def scatter(j, carry):
            idx = idx_v[pl.ds(j * _LANES, _LANES)]
            plsc.store_scatter(mask_v, [idx], zeros)
            return carry

        lax.fori_loop(0, n_pad // _LANES, scatter, 0)
        pltpu.sync_copy(mask_v, out_hbm)


def _build_mask_sc(idx_padded, n_cols):
    n_pad = idx_padded.shape[0]
    mesh = plsc.VectorSubcoreMesh(
        core_axis_name="c", subcore_axis_name="s", num_cores=2, num_subcores=16
    )
    k = pl.kernel(
        _mask_body,
        out_type=jax.ShapeDtypeStruct((n_cols,), jnp.float32),
        mesh=mesh,
        scratch_types=[
            pltpu.VMEM((n_pad,), jnp.int32),
            pltpu.VMEM((n_cols,), jnp.float32),
        ],
        compiler_params=pltpu.CompilerParams(needs_layout_passes=False),
    )
    return k(idx_padded)


def _mul_body(x_ref, m_ref, o_ref):
    o_ref[...] = x_ref[...] * m_ref[...]


_BM = 512


def kernel(x, mask_indices):
    m, n = x.shape
    n_idx = mask_indices.shape[0]
    n_pad = -(-n_idx // _LANES) * _LANES
    if n_pad != n_idx:
        pad = jnp.broadcast_to(mask_indices[0], (n_pad - n_idx,))
        idx_padded = jnp.concatenate([mask_indices, pad])
    else:
        idx_padded = mask_indices
    mask = _build_mask_sc(idx_padded, n)

    out = pl.pallas_call(
        _mul_body,
        grid=(m // _BM,),
        in_specs=[
            pl.BlockSpec((_BM, n), lambda i: (i, 0)),
            pl.BlockSpec((1, n), lambda i: (0, 0)),
        ],
        out_specs=pl.BlockSpec((_BM, n), lambda i: (i, 0)),
        out_shape=jax.ShapeDtypeStruct((m, n), x.dtype),
    )(x, mask.reshape(1, n))
    return out
